# baseline (speedup 1.0000x reference)
import functools

import jax
import jax.numpy as jnp
from jax import lax
from jax.experimental import pallas as pl
from jax.experimental.pallas import tpu as pltpu
from jax.experimental.pallas import tpu_sc as plsc

N = 10000
E = 320000
D = 128
H = 128
C = 40
G = 64

NC, NS, L = 2, 16, 16
NW = NC * NS
FPT = H // NW
EC = 20000
EPT = E // NW

_f32 = jnp.float32
_i32 = jnp.int32

_sc_mesh = plsc.VectorSubcoreMesh(
    core_axis_name="c", subcore_axis_name="s", num_cores=NC, num_subcores=NS)

_sc_params = pltpu.CompilerParams(needs_layout_passes=False)


def _wid():
    return lax.axis_index("s") * NC + lax.axis_index("c")


MASK16 = 0xFFFF


@functools.partial(
    pl.kernel,
    out_type=jax.ShapeDtypeStruct((NW, N), _f32),
    mesh=_sc_mesh,
    scratch_types=[
        pltpu.VMEM((N,), _f32),
        pltpu.VMEM((EPT,), _i32),
        pltpu.SemaphoreType.DMA,
    ],
    compiler_params=_sc_params,
)
def _deg_kernel(dst_hbm, out_hbm, hist, ebuf, sem):
    wid = _wid()
    zeros = jnp.zeros((L,), _f32)
    ones = jnp.full((L,), 1.0, _f32)

    cp = pltpu.async_copy(dst_hbm.at[pl.ds(wid * EPT, EPT)], ebuf, sem)

    def zero_body(i, _):
        hist[pl.ds(i * L, L)] = zeros
        return 0
    lax.fori_loop(0, N // L, zero_body, 0, unroll=8)

    cp.wait()

    @plsc.parallel_loop(0, EPT // L, 1, unroll=8)
    def grp(g):
        d = ebuf[pl.ds(g * L, L)]
        plsc.addupdate_scatter(hist, [d], ones)

    pltpu.sync_copy(hist, out_hbm.at[wid])


@functools.partial(
    pl.kernel,
    out_type=jax.ShapeDtypeStruct((H * N,), _f32),
    mesh=_sc_mesh,
    scratch_types=[
        pltpu.VMEM((FPT * N,), _f32),
        pltpu.VMEM((FPT * N,), _f32),
        pltpu.VMEM((EC,), _i32),
        pltpu.VMEM((EC,), _i32),
        pltpu.SemaphoreType.DMA,
        pltpu.SemaphoreType.DMA,
        pltpu.SemaphoreType.DMA,
    ],
    compiler_params=_sc_params,
)
def _scatter_kernel(feat_hbm, pk_hbm, out_hbm, feat, acc, ebuf0, ebuf1, semf,
                    sem0, sem1):
    wid = _wid()
    base = wid * FPT * N
    nch = E // EC
    ebufs = (ebuf0, ebuf1)
    sems = (sem0, sem1)

    cpf = pltpu.async_copy(feat_hbm.at[pl.ds(base, FPT * N)], feat, semf)
    pltpu.async_copy(pk_hbm.at[pl.ds(0, EC)], ebuf0, sem0)

    zeros = jnp.zeros((L,), _f32)

    def zero_body(i, _):
        acc[pl.ds(i * L, L)] = zeros
        return 0
    lax.fori_loop(0, FPT * N // L, zero_body, 0, unroll=8)

    cpf.wait()

    offs = [jnp.full((L,), f * N, _i32) for f in range(FPT)]

    def outer(cc, _):
        for b in range(2):
            c = cc * 2 + b
            pltpu.make_async_copy(
                pk_hbm.at[pl.ds(c * EC, EC)], ebufs[b], sems[b]).wait()

            @pl.when(c + 1 < nch)
            def _():
                pltpu.async_copy(pk_hbm.at[pl.ds((c + 1) * EC, EC)],
                                 ebufs[1 - b], sems[1 - b])

            ebc = ebufs[b]

            def grp(g, _):
                p = ebc[pl.ds(g * L, L)]
                s = p & MASK16
                d = lax.shift_right_logical(p, 16)
                for f in range(FPT):
                    v = plsc.load_gather(feat, [s + offs[f]])
                    plsc.addupdate_scatter(acc, [d + offs[f]], v)
                return 0
            lax.fori_loop(0, EC // L, grp, 0, unroll=8)
        return 0
    lax.fori_loop(0, nch // 2, outer, 0)

    pltpu.sync_copy(acc, out_hbm.at[pl.ds(base, FPT * N)])


_DOT = dict(preferred_element_type=_f32)


def _tc1_body(hists_ref, x_ref, w1_ref, src_ref, dst_ref, hs_ref, dinv_ref,
              pk_ref):
    deg = 1.0 + jnp.sum(hists_ref[...], axis=0, keepdims=True)
    dinv = lax.rsqrt(deg)
    hwT = lax.dot_general(w1_ref[...], x_ref[...],
                          (((0,), (1,)), ((), ())), **_DOT)
    hs_ref[...] = hwT * dinv
    dinv_ref[...] = dinv
    pk_ref[...] = src_ref[...] | (dst_ref[...] << 16)


def _bn_relu_T(pre, g_col, be_col):
    m = jnp.mean(pre, axis=1, keepdims=True)
    cen = pre - m
    var = jnp.mean(cen * cen, axis=1, keepdims=True)
    return jnp.maximum(cen * lax.rsqrt(var + 1e-5) * g_col + be_col, 0.0)


def _tc2_body(s1_ref, hs1_ref, dinv_ref, b1_ref, g1_ref, be1_ref, w2_ref,
              hs2_ref):
    dinv = dinv_ref[...]
    pre = (s1_ref[...] + hs1_ref[...]) * dinv + b1_ref[...]
    h1 = _bn_relu_T(pre, g1_ref[...], be1_ref[...])
    hw2 = lax.dot_general(w2_ref[...], h1, (((0,), (0,)), ((), ())), **_DOT)
    hs2_ref[...] = hw2 * dinv


def _tc3_body(s2_ref, hs2_ref, dinv_ref, b2_ref, g2_ref, be2_ref, batch_ref,
              fcw_ref, fcb_ref, out_ref):
    dinv = dinv_ref[...]
    pre = (s2_ref[...] + hs2_ref[...]) * dinv + b2_ref[...]
    h2 = _bn_relu_T(pre, g2_ref[...], be2_ref[...])
    seg = lax.broadcasted_iota(_i32, (G, N), 0)
    onehot = (batch_ref[...] == seg).astype(_f32)
    cnt = jnp.sum(onehot, axis=1, keepdims=True)
    pooled = lax.dot_general(onehot, h2, (((1,), (1,)), ((), ())), **_DOT)
    pooled = pooled / jnp.maximum(cnt, 1.0)
    out_ref[...] = lax.dot_general(pooled, fcw_ref[...],
                                   (((1,), (0,)), ((), ())), **_DOT) \
        + fcb_ref[...]


_tc1 = pl.pallas_call(
    _tc1_body,
    out_shape=[jax.ShapeDtypeStruct((H, N), _f32),
               jax.ShapeDtypeStruct((1, N), _f32),
               jax.ShapeDtypeStruct((E // 640, 640), _i32)],
)

_tc2 = pl.pallas_call(
    _tc2_body,
    out_shape=jax.ShapeDtypeStruct((H, N), _f32),
)

_tc3 = pl.pallas_call(
    _tc3_body,
    out_shape=jax.ShapeDtypeStruct((G, C), _f32),
)


def kernel(x, edge_index, batch, W1, b1, g1, be1, W2, b2, g2, be2, fcW, fcb):
    src = edge_index[0]
    dst = edge_index[1]
    hists = _deg_kernel(dst)
    hs1T, dinv, packed2 = _tc1(hists, x, W1, src.reshape(E // 640, 640),
                               dst.reshape(E // 640, 640))
    packed = packed2.reshape(E)
    s1T = _scatter_kernel(hs1T.reshape(H * N), packed).reshape(H, N)
    hs2T = _tc2(s1T, hs1T, dinv, b1.reshape(H, 1), g1.reshape(H, 1),
                be1.reshape(H, 1), W2)
    s2T = _scatter_kernel(hs2T.reshape(H * N), packed).reshape(H, N)
    out = _tc3(s2T, hs2T, dinv, b2.reshape(H, 1), g2.reshape(H, 1),
               be2.reshape(H, 1), batch.reshape(1, N), fcW,
               fcb.reshape(1, C))
    return out


# speedup vs baseline: 2.4839x; 2.4839x over previous
import functools

import jax
import jax.numpy as jnp
from jax import lax
from jax.experimental import pallas as pl
from jax.experimental.pallas import tpu as pltpu
from jax.experimental.pallas import tpu_sc as plsc

N = 10000
E = 320000
D = 128
H = 128
C = 40
G = 64

NC, NS, L = 2, 16, 16
NW = NC * NS
FPT = H // NW
EC = 20000
EPT = E // NW

_f32 = jnp.float32
_i32 = jnp.int32

_sc_mesh = plsc.VectorSubcoreMesh(
    core_axis_name="c", subcore_axis_name="s", num_cores=NC, num_subcores=NS)

_sc_params = pltpu.CompilerParams(needs_layout_passes=False)


def _wid():
    return lax.axis_index("s") * NC + lax.axis_index("c")


MASK16 = 0xFFFF


@functools.partial(
    pl.kernel,
    out_type=jax.ShapeDtypeStruct((NW, N), _f32),
    mesh=_sc_mesh,
    scratch_types=[
        pltpu.VMEM((N,), _f32),
        pltpu.VMEM((EPT,), _i32),
        pltpu.SemaphoreType.DMA,
    ],
    compiler_params=_sc_params,
)
def _deg_kernel(dst_hbm, out_hbm, hist, ebuf, sem):
    wid = _wid()
    zeros = jnp.zeros((L,), _f32)
    ones = jnp.full((L,), 1.0, _f32)

    cp = pltpu.async_copy(dst_hbm.at[pl.ds(wid * EPT, EPT)], ebuf, sem)

    def zero_body(i, _):
        hist[pl.ds(i * L, L)] = zeros
        return 0
    lax.fori_loop(0, N // L, zero_body, 0, unroll=8)

    cp.wait()

    @plsc.parallel_loop(0, EPT // L, 1, unroll=8)
    def grp(g):
        d = ebuf[pl.ds(g * L, L)]
        plsc.addupdate_scatter(hist, [d], ones)

    pltpu.sync_copy(hist, out_hbm.at[wid])


@functools.partial(
    pl.kernel,
    out_type=jax.ShapeDtypeStruct((H * N,), _f32),
    mesh=_sc_mesh,
    scratch_types=[
        pltpu.VMEM((FPT * N,), _f32),
        pltpu.VMEM((FPT * N,), _f32),
        pltpu.VMEM((EC,), _i32),
        pltpu.VMEM((EC,), _i32),
        pltpu.SemaphoreType.DMA,
        pltpu.SemaphoreType.DMA,
        pltpu.SemaphoreType.DMA,
    ],
    compiler_params=_sc_params,
)
def _scatter_kernel(feat_hbm, pk_hbm, out_hbm, feat, acc, ebuf0, ebuf1, semf,
                    sem0, sem1):
    wid = _wid()
    base = wid * FPT * N
    nch = E // EC
    ebufs = (ebuf0, ebuf1)
    sems = (sem0, sem1)

    cpf = pltpu.async_copy(feat_hbm.at[pl.ds(base, FPT * N)], feat, semf)
    pltpu.async_copy(pk_hbm.at[pl.ds(0, EC)], ebuf0, sem0)

    zeros = jnp.zeros((L,), _f32)

    def zero_body(i, _):
        acc[pl.ds(i * L, L)] = zeros
        return 0
    lax.fori_loop(0, FPT * N // L, zero_body, 0, unroll=8)

    cpf.wait()

    offs = [jnp.full((L,), f * N, _i32) for f in range(FPT)]

    def outer(cc, _):
        for b in range(2):
            c = cc * 2 + b
            pltpu.make_async_copy(
                pk_hbm.at[pl.ds(c * EC, EC)], ebufs[b], sems[b]).wait()

            @pl.when(c + 1 < nch)
            def _():
                pltpu.async_copy(pk_hbm.at[pl.ds((c + 1) * EC, EC)],
                                 ebufs[1 - b], sems[1 - b])

            ebc = ebufs[b]

            @plsc.parallel_loop(0, EC // L, 1, unroll=8)
            def grp(g):
                p = ebc[pl.ds(g * L, L)]
                s = p & MASK16
                d = lax.shift_right_logical(p, 16)
                for f in range(FPT):
                    v = plsc.load_gather(feat, [s + offs[f]])
                    plsc.addupdate_scatter(acc, [d + offs[f]], v)
        return 0
    lax.fori_loop(0, nch // 2, outer, 0)

    pltpu.sync_copy(acc, out_hbm.at[pl.ds(base, FPT * N)])


_DOT = dict(preferred_element_type=_f32)


def _tc1_body(hists_ref, x_ref, w1_ref, src_ref, dst_ref, hs_ref, dinv_ref,
              pk_ref):
    deg = 1.0 + jnp.sum(hists_ref[...], axis=0, keepdims=True)
    dinv = lax.rsqrt(deg)
    hwT = lax.dot_general(w1_ref[...], x_ref[...],
                          (((0,), (1,)), ((), ())), **_DOT)
    hs_ref[...] = hwT * dinv
    dinv_ref[...] = dinv
    pk_ref[...] = src_ref[...] | (dst_ref[...] << 16)


def _bn_relu_T(pre, g_col, be_col):
    m = jnp.mean(pre, axis=1, keepdims=True)
    cen = pre - m
    var = jnp.mean(cen * cen, axis=1, keepdims=True)
    return jnp.maximum(cen * lax.rsqrt(var + 1e-5) * g_col + be_col, 0.0)


def _tc2_body(s1_ref, hs1_ref, dinv_ref, b1_ref, g1_ref, be1_ref, w2_ref,
              hs2_ref):
    dinv = dinv_ref[...]
    pre = (s1_ref[...] + hs1_ref[...]) * dinv + b1_ref[...]
    h1 = _bn_relu_T(pre, g1_ref[...], be1_ref[...])
    hw2 = lax.dot_general(w2_ref[...], h1, (((0,), (0,)), ((), ())), **_DOT)
    hs2_ref[...] = hw2 * dinv


def _tc3_body(s2_ref, hs2_ref, dinv_ref, b2_ref, g2_ref, be2_ref, batch_ref,
              fcw_ref, fcb_ref, out_ref):
    dinv = dinv_ref[...]
    pre = (s2_ref[...] + hs2_ref[...]) * dinv + b2_ref[...]
    h2 = _bn_relu_T(pre, g2_ref[...], be2_ref[...])
    seg = lax.broadcasted_iota(_i32, (G, N), 0)
    onehot = (batch_ref[...] == seg).astype(_f32)
    cnt = jnp.sum(onehot, axis=1, keepdims=True)
    pooled = lax.dot_general(onehot, h2, (((1,), (1,)), ((), ())), **_DOT)
    pooled = pooled / jnp.maximum(cnt, 1.0)
    out_ref[...] = lax.dot_general(pooled, fcw_ref[...],
                                   (((1,), (0,)), ((), ())), **_DOT) \
        + fcb_ref[...]


_tc1 = pl.pallas_call(
    _tc1_body,
    out_shape=[jax.ShapeDtypeStruct((H, N), _f32),
               jax.ShapeDtypeStruct((1, N), _f32),
               jax.ShapeDtypeStruct((E // 640, 640), _i32)],
)

_tc2 = pl.pallas_call(
    _tc2_body,
    out_shape=jax.ShapeDtypeStruct((H, N), _f32),
)

_tc3 = pl.pallas_call(
    _tc3_body,
    out_shape=jax.ShapeDtypeStruct((G, C), _f32),
)


def kernel(x, edge_index, batch, W1, b1, g1, be1, W2, b2, g2, be2, fcW, fcb):
    src = edge_index[0]
    dst = edge_index[1]
    hists = _deg_kernel(dst)
    hs1T, dinv, packed2 = _tc1(hists, x, W1, src.reshape(E // 640, 640),
                               dst.reshape(E // 640, 640))
    packed = packed2.reshape(E)
    s1T = _scatter_kernel(hs1T.reshape(H * N), packed).reshape(H, N)
    hs2T = _tc2(s1T, hs1T, dinv, b1.reshape(H, 1), g1.reshape(H, 1),
                be1.reshape(H, 1), W2)
    s2T = _scatter_kernel(hs2T.reshape(H * N), packed).reshape(H, N)
    out = _tc3(s2T, hs2T, dinv, b2.reshape(H, 1), g2.reshape(H, 1),
               be2.reshape(H, 1), batch.reshape(1, N), fcW,
               fcb.reshape(1, C))
    return out
